# baseline (speedup 1.0000x reference)
import functools

import jax
import jax.numpy as jnp
from jax import lax
from jax.experimental import pallas as pl
from jax.experimental.pallas import tpu as pltpu
from jax.experimental.pallas import tpu_sc as plsc

_B = 16384
_D = 128
_E = 8

_BSC = 8192
_NC = 2
_NS = 16
_NW = _NC * _NS
_BPW = _BSC // _NW
_NCH = 4
_CH = _BPW // _NCH

_BTC = _B - _BSC
_TBLK = 2048


def _emotion_lookup_sc(table_hbm, idx_hbm, out_hbm, idx_v, rows_v, tbl_sh,
                       gsem, osem):
    sid = lax.axis_index("s")
    wid = sid * _NC + lax.axis_index("c")
    base = wid * _BPW

    @pl.when(sid == 0)
    def _():
        pltpu.sync_copy(table_hbm, tbl_sh)

    pltpu.sync_copy(idx_hbm.at[pl.ds(base, _BPW)], idx_v)
    plsc.subcore_barrier()

    gathers = []
    for k in range(_NCH):
        gathers.append(
            pltpu.async_copy(tbl_sh.at[idx_v.at[pl.ds(k * _CH, _CH)]],
                             rows_v.at[k], gsem))
    outs = []
    for k in range(_NCH):
        gathers[k].wait()
        outs.append(
            pltpu.async_copy(rows_v.at[k],
                             out_hbm.at[pl.ds(base + k * _CH, _CH)], osem))
    for k in range(_NCH):
        outs[k].wait()


def _emotion_lookup_tc(idx_ref, tbl_ref, out_ref):
    onehot = (idx_ref[...] == lax.broadcasted_iota(jnp.int32, (_TBLK, _E), 1)
              ).astype(jnp.float32)
    out_ref[...] = jnp.dot(onehot, tbl_ref[...],
                           preferred_element_type=jnp.float32)


@jax.jit
def kernel(emotion_id, embedding_weight):
    idx = emotion_id.astype(jnp.int32)

    mesh = plsc.VectorSubcoreMesh(core_axis_name="c", subcore_axis_name="s")
    sc_run = functools.partial(
        pl.kernel,
        mesh=mesh,
        out_type=jax.ShapeDtypeStruct((_BSC, _D), jnp.float32),
        scratch_types=[
            pltpu.VMEM((_BPW,), jnp.int32),
            pltpu.VMEM((_NCH, _CH, _D), jnp.float32),
            pltpu.VMEM_SHARED((_E, _D), jnp.float32),
            pltpu.SemaphoreType.DMA,
            pltpu.SemaphoreType.DMA,
        ],
    )(_emotion_lookup_sc)
    out_sc = sc_run(embedding_weight, idx[:_BSC])

    out_tc = pl.pallas_call(
        _emotion_lookup_tc,
        grid=(_BTC // _TBLK,),
        in_specs=[
            pl.BlockSpec((_TBLK, 1), lambda i: (i, 0)),
            pl.BlockSpec((_E, _D), lambda i: (0, 0)),
        ],
        out_specs=pl.BlockSpec((_TBLK, _D), lambda i: (i, 0)),
        out_shape=jax.ShapeDtypeStruct((_BTC, _D), jnp.float32),
    )(idx[_BSC:].reshape(_BTC, 1), embedding_weight)

    return jnp.concatenate([out_sc, out_tc], axis=0)


# speedup vs baseline: 1.3080x; 1.3080x over previous
import functools

import jax
import jax.numpy as jnp
from jax import lax
from jax.experimental import pallas as pl
from jax.experimental.pallas import tpu as pltpu
from jax.experimental.pallas import tpu_sc as plsc

_B = 16384
_D = 128
_E = 8
_NC = 2
_NS = 16
_NW = _NC * _NS
_BPW = _B // _NW
_NCH = 8
_CH = _BPW // _NCH


def _emotion_lookup_sc(table_hbm, idx_hbm, out_hbm, idx_v, rows_v, tbl_sh,
                       tsem, isem, gsem, osem):
    sid = lax.axis_index("s")
    wid = sid * _NC + lax.axis_index("c")
    base = wid * _BPW

    tcp = pltpu.async_copy(table_hbm, tbl_sh.at[sid], tsem)
    icp = pltpu.async_copy(idx_hbm.at[pl.ds(base, _BPW)], idx_v, isem)
    tcp.wait()
    icp.wait()

    gathers = []
    for k in range(_NCH):
        gathers.append(
            pltpu.async_copy(tbl_sh.at[sid].at[idx_v.at[pl.ds(k * _CH, _CH)]],
                             rows_v.at[k], gsem))
    outs = []
    for k in range(_NCH):
        gathers[k].wait()
        outs.append(
            pltpu.async_copy(rows_v.at[k],
                             out_hbm.at[pl.ds(base + k * _CH, _CH)], osem))
    for k in range(_NCH):
        outs[k].wait()


@jax.jit
def kernel(emotion_id, embedding_weight):
    mesh = plsc.VectorSubcoreMesh(core_axis_name="c", subcore_axis_name="s")
    run = functools.partial(
        pl.kernel,
        mesh=mesh,
        out_type=jax.ShapeDtypeStruct((_B, _D), jnp.float32),
        scratch_types=[
            pltpu.VMEM((_BPW,), jnp.int32),
            pltpu.VMEM((_NCH, _CH, _D), jnp.float32),
            pltpu.VMEM_SHARED((_NS, _E, _D), jnp.float32),
            pltpu.SemaphoreType.DMA,
            pltpu.SemaphoreType.DMA,
            pltpu.SemaphoreType.DMA,
            pltpu.SemaphoreType.DMA,
        ],
    )(_emotion_lookup_sc)
    return run(embedding_weight, emotion_id.astype(jnp.int32))


# speedup vs baseline: 1.3262x; 1.0139x over previous
import functools

import jax
import jax.numpy as jnp
from jax import lax
from jax.experimental import pallas as pl
from jax.experimental.pallas import tpu as pltpu
from jax.experimental.pallas import tpu_sc as plsc

_B = 16384
_D = 128
_NC = 2
_NS = 16
_NW = _NC * _NS
_BPW = _B // _NW
_NCH = 4
_CH = _BPW // _NCH


def _emotion_lookup_sc(table_hbm, idx_hbm, out_hbm, idx_v, rows_v, tbl_sh,
                       gsem, osem):
    sid = lax.axis_index("s")
    wid = sid * _NC + lax.axis_index("c")
    base = wid * _BPW

    @pl.when(sid == 0)
    def _():
        pltpu.sync_copy(table_hbm, tbl_sh)

    pltpu.sync_copy(idx_hbm.at[pl.ds(base, _BPW)], idx_v)
    plsc.subcore_barrier()

    gathers = []
    for k in range(_NCH):
        gathers.append(
            pltpu.async_copy(tbl_sh.at[idx_v.at[pl.ds(k * _CH, _CH)]],
                             rows_v.at[k], gsem))
    outs = []
    for k in range(_NCH):
        gathers[k].wait()
        outs.append(
            pltpu.async_copy(rows_v.at[k],
                             out_hbm.at[pl.ds(base + k * _CH, _CH)], osem))
    for k in range(_NCH):
        outs[k].wait()


@jax.jit
def kernel(emotion_id, embedding_weight):
    mesh = plsc.VectorSubcoreMesh(core_axis_name="c", subcore_axis_name="s")
    run = functools.partial(
        pl.kernel,
        mesh=mesh,
        out_type=jax.ShapeDtypeStruct((_B, _D), jnp.float32),
        scratch_types=[
            pltpu.VMEM((_BPW,), jnp.int32),
            pltpu.VMEM((_NCH, _CH, _D), jnp.float32),
            pltpu.VMEM_SHARED((8, _D), jnp.float32),
            pltpu.SemaphoreType.DMA,
            pltpu.SemaphoreType.DMA,
        ],
    )(_emotion_lookup_sc)
    return run(embedding_weight, emotion_id.astype(jnp.int32))
